# Initial kernel scaffold; baseline (speedup 1.0000x reference)
import jax
import jax.numpy as jnp
from jax.experimental import pallas as pl


def kernel(feature, edge_index, W, b):
    raise NotImplementedError("write your pallas kernel here")


# speedup vs baseline: 5.6155x; 5.6155x over previous
import functools

import jax
import jax.numpy as jnp
from jax import lax
from jax.experimental import pallas as pl
from jax.experimental.pallas import tpu as pltpu
from jax.experimental.pallas import tpu_sc as plsc

N_NODES = 10000
N_PAD = 10240
E = 320000
D = 128
NC, NS = 2, 16
NW = NC * NS
EPW = E // NW
CH = 128
NCH = -(-EPW // CH)
EPW_PAD = NCH * CH
ROWS_PT = N_PAD // NS
BR = 2000


def _sc_aggregate(src3, dst3, feature, zeros2d, zeros1d, ones1):
    mesh = plsc.VectorSubcoreMesh(core_axis_name="c", subcore_axis_name="s")

    @functools.partial(
        pl.kernel,
        mesh=mesh,
        out_type=[
            jax.ShapeDtypeStruct((NC, N_PAD, D), jnp.float32),
            jax.ShapeDtypeStruct((NC, N_PAD), jnp.float32),
        ],
        scratch_types=[
            pltpu.VMEM((NCH, CH), jnp.int32),
            pltpu.VMEM((NCH, CH), jnp.int32),
            pltpu.VMEM((CH, D), jnp.float32),
            pltpu.VMEM((CH,), jnp.float32),
            pltpu.VMEM_SHARED((N_PAD, D), jnp.float32),
            pltpu.VMEM_SHARED((N_PAD,), jnp.float32),
            pltpu.SemaphoreType.DMA,
        ],
    )
    def k(src_hbm, dst_hbm, feat_hbm, z2_hbm, z1_hbm, ones_hbm,
          agg_out, deg_out, src_v, dst_v, rows_v, ones_v, agg_sh, deg_sh, sem):
        c = lax.axis_index("c")
        s = lax.axis_index("s")
        w = c * NS + s
        pltpu.sync_copy(src_hbm.at[w], src_v)
        pltpu.sync_copy(dst_hbm.at[w], dst_v)
        pltpu.sync_copy(ones_hbm, ones_v)
        base = s * ROWS_PT
        pltpu.sync_copy(z2_hbm, agg_sh.at[pl.ds(base, ROWS_PT)])
        pltpu.sync_copy(z1_hbm, deg_sh.at[pl.ds(base, ROWS_PT)])
        plsc.subcore_barrier()

        def body(j, carry):
            pltpu.async_copy(feat_hbm.at[src_v.at[j]], rows_v, sem).wait()
            pltpu.sync_copy(rows_v, agg_sh.at[dst_v.at[j]], add=True)
            pltpu.sync_copy(ones_v, deg_sh.at[dst_v.at[j]], add=True)
            return carry

        lax.fori_loop(0, NCH, body, 0)
        plsc.subcore_barrier()
        pltpu.sync_copy(agg_sh.at[pl.ds(base, ROWS_PT)],
                        agg_out.at[c, pl.ds(base, ROWS_PT)])
        pltpu.sync_copy(deg_sh.at[pl.ds(base, ROWS_PT)],
                        deg_out.at[c, pl.ds(base, ROWS_PT)])

    return k(src3, dst3, feature, zeros2d, zeros1d, ones1)


def _tc_body(f_ref, a_ref, d_ref, w_ref, b_ref, o_ref):
    f = f_ref[...]
    agg = a_ref[0] + a_ref[1]
    deg = jnp.maximum(d_ref[0, 0] + d_ref[0, 1], 1.0)
    nf = agg / deg[:, None]
    w1 = w_ref[0:D, :]
    w2 = w_ref[D:2 * D, :]
    bb = b_ref[0, :]
    t2 = jnp.dot(nf, w2, preferred_element_type=jnp.float32) + bb[None, :]
    o1 = jnp.dot(f, w1, preferred_element_type=jnp.float32) + t2
    o2 = jnp.dot(o1, w1, preferred_element_type=jnp.float32) + t2
    o_ref[...] = jnp.maximum(o2, 0.0)


def _tc_combine(feature, agg2, degt, W, b2):
    return pl.pallas_call(
        _tc_body,
        grid=(N_NODES // BR,),
        in_specs=[
            pl.BlockSpec((BR, D), lambda i: (i, 0)),
            pl.BlockSpec((NC, BR, D), lambda i: (0, i, 0)),
            pl.BlockSpec((1, NC, BR), lambda i: (i, 0, 0)),
            pl.BlockSpec((2 * D, D), lambda i: (0, 0)),
            pl.BlockSpec((1, D), lambda i: (0, 0)),
        ],
        out_specs=pl.BlockSpec((BR, D), lambda i: (i, 0)),
        out_shape=jax.ShapeDtypeStruct((N_NODES, D), jnp.float32),
    )(feature, agg2, degt, W, b2)


def kernel(feature, edge_index, W, b):
    src = edge_index[0].astype(jnp.int32)
    dst = edge_index[1].astype(jnp.int32)
    pad = NW * EPW_PAD - E
    src3 = jnp.concatenate([src, jnp.zeros((pad,), jnp.int32)]).reshape(NW, NCH, CH)
    dst3 = jnp.concatenate([dst, jnp.full((pad,), N_NODES, jnp.int32)]).reshape(NW, NCH, CH)
    zeros2d = jnp.zeros((ROWS_PT, D), jnp.float32)
    zeros1d = jnp.zeros((ROWS_PT,), jnp.float32)
    ones1 = jnp.ones((CH,), jnp.float32)
    agg2, deg2 = _sc_aggregate(src3, dst3, feature, zeros2d, zeros1d, ones1)
    degt = deg2[:, :N_NODES].reshape(NC, N_NODES // BR, BR).transpose(1, 0, 2)
    return _tc_combine(feature, agg2, degt, W, b.reshape(1, D))
